# Initial kernel scaffold; baseline (speedup 1.0000x reference)
import jax
import jax.numpy as jnp
from jax.experimental import pallas as pl


def kernel(locations, levels, batch_indices, box_cls, box_regression, centerness, image_sizes):
    raise NotImplementedError("write your pallas kernel here")


# speedup vs baseline: 2.8684x; 2.8684x over previous
import functools

import jax
import jax.numpy as jnp
from jax.experimental import pallas as pl

_PRE_NMS_THRESH = 0.05
_PRE_NMS_TOP_N = 1000
_NMS_THRESH = 0.6
_POST_TOP_N = 100
_STRIDES = (8.0, 16.0, 32.0, 64.0, 128.0)
_PADK = 1024


def _score_body(bc_ref, cn_ref, bi_ref, o_ref, *, C):
    i = pl.program_id(0)
    bc = bc_ref[...]
    cn = cn_ref[...]
    bi = bi_ref[...]
    cls = (1.0 / (1.0 + jnp.exp(-bc))) * (1.0 / (1.0 + jnp.exp(-cn)))
    col = jax.lax.broadcasted_iota(jnp.int32, bc.shape, 1)
    valid = (cls > _PRE_NMS_THRESH) & (col < (C - 1)) & (bi == i)
    o_ref[...] = jnp.where(valid, jnp.sqrt(cls), -1.0)[None]


def _nms_body(lx_ref, ly_ref, r0_ref, r1_ref, r2_ref, r3_ref, v_ref, cl_ref,
              hw_ref, o_ref):
    lx = lx_ref[0]
    ly = ly_ref[0]
    r0 = r0_ref[0]
    r1 = r1_ref[0]
    r2 = r2_ref[0]
    r3 = r3_ref[0]
    v = v_ref[0]
    cl = cl_ref[0]
    h = hw_ref[0, 0, 0]
    w = hw_ref[0, 0, 1]

    x1 = jnp.clip(lx - r0, 0.0, w - 1.0)
    y1 = jnp.clip(ly - r1, 0.0, h - 1.0)
    x2 = jnp.clip(lx + r2, 0.0, w - 1.0)
    y2 = jnp.clip(ly + r3, 0.0, h - 1.0)
    ws = x2 - x1 + 1.0
    hs = y2 - y1 + 1.0
    valid = (v > 0.0) & (ws >= 0.0) & (hs >= 0.0)

    neg = jnp.float32(-jnp.inf)

    def mmax(a):
        return jnp.max(jnp.where(valid, a, neg))

    mx = jnp.maximum(jnp.maximum(mmax(x1), mmax(y1)),
                     jnp.maximum(mmax(x2), mmax(y2)))
    mx = jnp.where(mx == neg, 0.0, mx)
    off = cl * (mx + 1.0)
    bx1 = x1 + off
    by1 = y1 + off
    bx2 = x2 + off
    by2 = y2 + off
    areas = (bx2 - bx1 + 1.0) * (by2 - by1 + 1.0)

    ar = jax.lax.broadcasted_iota(jnp.int32, (1, _PADK), 1)

    validf = jnp.where(valid, 1.0, 0.0)

    def body(j, suppf):
        sel = ar == j
        self_f = jnp.where(sel, 1.0, 0.0)
        bx1j = jnp.sum(self_f * bx1)
        by1j = jnp.sum(self_f * by1)
        bx2j = jnp.sum(self_f * bx2)
        by2j = jnp.sum(self_f * by2)
        aj = jnp.sum(self_f * areas)
        keepf = jnp.sum(self_f * validf * (1.0 - suppf))
        xx1 = jnp.maximum(bx1j, bx1)
        yy1 = jnp.maximum(by1j, by1)
        xx2 = jnp.minimum(bx2j, bx2)
        yy2 = jnp.minimum(by2j, by2)
        iw = jnp.maximum(0.0, xx2 - xx1 + 1.0)
        ih = jnp.maximum(0.0, yy2 - yy1 + 1.0)
        inter = iw * ih
        iou = inter / (aj + areas - inter)
        newsf = jnp.where((iou > _NMS_THRESH) & (ar > j), 1.0, 0.0)
        return jnp.where(keepf > 0.5, jnp.maximum(suppf, newsf), suppf)

    suppf = jax.lax.fori_loop(0, _PRE_NMS_TOP_N, body,
                              jnp.zeros((1, _PADK), jnp.float32))

    kept = valid & (suppf < 0.5)
    c = jnp.where(kept, 1.0, 0.0)
    d = 1
    while d < _PADK:
        c = c + jnp.concatenate(
            [jnp.zeros((1, d), jnp.float32), c[:, :_PADK - d]], axis=1)
        d *= 2
    rank = (c - 1.0).astype(jnp.int32)
    final = kept & (rank < _POST_TOP_N)
    rows = jax.lax.broadcasted_iota(jnp.int32, (128, _PADK), 0)
    ohf = jnp.where((rows == rank) & final, 1.0, 0.0)
    vmat = jnp.concatenate(
        [x1, y1, x2, y2, v, cl, jnp.zeros((2, _PADK), jnp.float32)], axis=0)
    out = jax.lax.dot_general(ohf, vmat, (((1,), (1,)), ((), ())),
                              preferred_element_type=jnp.float32)
    o_ref[...] = out[None]


def kernel(locations, levels, batch_indices, box_cls, box_regression,
           centerness, image_sizes):
    m, C = box_cls.shape
    N = image_sizes.shape[0]
    P = 128
    R = 2000

    bc = jnp.pad(box_cls.astype(jnp.float32), ((0, 0), (0, P - C)),
                 constant_values=-1e9)
    cn = centerness.astype(jnp.float32).reshape(m, 1)
    bi = batch_indices.astype(jnp.int32).reshape(m, 1)

    scored = pl.pallas_call(
        functools.partial(_score_body, C=C),
        grid=(N, m // R),
        in_specs=[
            pl.BlockSpec((R, P), lambda i, rb: (rb, 0)),
            pl.BlockSpec((R, 1), lambda i, rb: (rb, 0)),
            pl.BlockSpec((R, 1), lambda i, rb: (rb, 0)),
        ],
        out_specs=pl.BlockSpec((1, R, P), lambda i, rb: (i, rb, 0)),
        out_shape=jax.ShapeDtypeStruct((N, m, P), jnp.float32),
    )(bc, cn, bi)

    vals_k, idx_k = jax.lax.top_k(scored.reshape(N, m * P), _PRE_NMS_TOP_N)
    rsel = idx_k // P
    csel = idx_k % P

    strides = jnp.asarray(_STRIDES, jnp.float32)
    reg = box_regression.astype(jnp.float32) * strides[levels][:, None]
    locs = locations.astype(jnp.float32)

    pad = _PADK - _PRE_NMS_TOP_N
    def padz(a, cv=0.0):
        return jnp.pad(a, ((0, 0), (0, pad)),
                       constant_values=cv).reshape(N, 1, _PADK)

    lx = padz(locs[rsel, 0])
    ly = padz(locs[rsel, 1])
    r0 = padz(reg[rsel, 0])
    r1 = padz(reg[rsel, 1])
    r2 = padz(reg[rsel, 2])
    r3 = padz(reg[rsel, 3])
    vp = padz(vals_k, cv=-1.0)
    clab = padz((csel + 1).astype(jnp.float32))
    hw = jnp.zeros((N, 1, 128), jnp.float32)
    hw = hw.at[:, 0, 0].set(image_sizes[:, 0].astype(jnp.float32))
    hw = hw.at[:, 0, 1].set(image_sizes[:, 1].astype(jnp.float32))

    vec = pl.BlockSpec((1, 1, _PADK), lambda i: (i, 0, 0))
    out = pl.pallas_call(
        _nms_body,
        grid=(N,),
        in_specs=[vec, vec, vec, vec, vec, vec, vec, vec,
                  pl.BlockSpec((1, 1, 128), lambda i: (i, 0, 0))],
        out_specs=pl.BlockSpec((1, 128, 8), lambda i: (i, 0, 0)),
        out_shape=jax.ShapeDtypeStruct((N, 128, 8), jnp.float32),
    )(lx, ly, r0, r1, r2, r3, vp, clab, hw)

    vals5 = out[:, :_POST_TOP_N, :5].reshape(N * _POST_TOP_N, 5)
    labels = jnp.rint(out[:, :_POST_TOP_N, 5]).astype(jnp.int32).reshape(-1)
    return vals5, labels


# speedup vs baseline: 4.7951x; 1.6717x over previous
import functools

import jax
import jax.numpy as jnp
from jax.experimental import pallas as pl

_PRE_NMS_THRESH = 0.05
_PRE_NMS_TOP_N = 1000
_NMS_THRESH = 0.6
_POST_TOP_N = 100
_STRIDES = (8.0, 16.0, 32.0, 64.0, 128.0)
_PADK = 1024


def _score_body(bc_ref, cn_ref, bi_ref, o_ref, *, C):
    i = pl.program_id(0)
    bc = bc_ref[...]
    cn = cn_ref[...]
    bi = bi_ref[...]
    cls = (1.0 / (1.0 + jnp.exp(-bc))) * (1.0 / (1.0 + jnp.exp(-cn)))
    col = jax.lax.broadcasted_iota(jnp.int32, bc.shape, 1)
    valid = (cls > _PRE_NMS_THRESH) & (col < (C - 1)) & (bi == i)
    o_ref[...] = jnp.where(valid, jnp.sqrt(cls), -1.0)[None]


def _nms_body(lx_ref, ly_ref, r0_ref, r1_ref, r2_ref, r3_ref, v_ref, cl_ref,
              hw_ref, o_ref):
    lx = lx_ref[0]
    ly = ly_ref[0]
    r0 = r0_ref[0]
    r1 = r1_ref[0]
    r2 = r2_ref[0]
    r3 = r3_ref[0]
    v = v_ref[0]
    cl = cl_ref[0]
    h = hw_ref[0, 0, 0]
    w = hw_ref[0, 0, 1]

    x1 = jnp.clip(lx - r0, 0.0, w - 1.0)
    y1 = jnp.clip(ly - r1, 0.0, h - 1.0)
    x2 = jnp.clip(lx + r2, 0.0, w - 1.0)
    y2 = jnp.clip(ly + r3, 0.0, h - 1.0)
    ws = x2 - x1 + 1.0
    hs = y2 - y1 + 1.0
    valid = (v > 0.0) & (ws >= 0.0) & (hs >= 0.0)

    neg = jnp.float32(-jnp.inf)

    def mmax(a):
        return jnp.max(jnp.where(valid, a, neg))

    mx = jnp.maximum(jnp.maximum(mmax(x1), mmax(y1)),
                     jnp.maximum(mmax(x2), mmax(y2)))
    mx = jnp.where(mx == neg, 0.0, mx)
    off = cl * (mx + 1.0)
    bx1 = x1 + off
    by1 = y1 + off
    bx2 = x2 + off
    by2 = y2 + off
    areas = (bx2 - bx1 + 1.0) * (by2 - by1 + 1.0)

    ar = jax.lax.broadcasted_iota(jnp.int32, (1, _PADK), 1)

    validf = jnp.where(valid, 1.0, 0.0)

    def body(j, suppf):
        sel = ar == j
        self_f = jnp.where(sel, 1.0, 0.0)
        bx1j = jnp.sum(self_f * bx1)
        by1j = jnp.sum(self_f * by1)
        bx2j = jnp.sum(self_f * bx2)
        by2j = jnp.sum(self_f * by2)
        aj = jnp.sum(self_f * areas)
        keepf = jnp.sum(self_f * validf * (1.0 - suppf))
        xx1 = jnp.maximum(bx1j, bx1)
        yy1 = jnp.maximum(by1j, by1)
        xx2 = jnp.minimum(bx2j, bx2)
        yy2 = jnp.minimum(by2j, by2)
        iw = jnp.maximum(0.0, xx2 - xx1 + 1.0)
        ih = jnp.maximum(0.0, yy2 - yy1 + 1.0)
        inter = iw * ih
        iou = inter / (aj + areas - inter)
        newsf = jnp.where((iou > _NMS_THRESH) & (ar > j), 1.0, 0.0)
        return jnp.where(keepf > 0.5, jnp.maximum(suppf, newsf), suppf)

    suppf = jax.lax.fori_loop(0, _PRE_NMS_TOP_N, body,
                              jnp.zeros((1, _PADK), jnp.float32))

    kept = valid & (suppf < 0.5)
    c = jnp.where(kept, 1.0, 0.0)
    d = 1
    while d < _PADK:
        c = c + jnp.concatenate(
            [jnp.zeros((1, d), jnp.float32), c[:, :_PADK - d]], axis=1)
        d *= 2
    rank = (c - 1.0).astype(jnp.int32)
    final = kept & (rank < _POST_TOP_N)
    rows = jax.lax.broadcasted_iota(jnp.int32, (128, _PADK), 0)
    ohf = jnp.where((rows == rank) & final, 1.0, 0.0)
    vmat = jnp.concatenate(
        [x1, y1, x2, y2, v, cl, jnp.zeros((2, _PADK), jnp.float32)], axis=0)
    out = jax.lax.dot_general(ohf, vmat, (((1,), (1,)), ((), ())),
                              preferred_element_type=jnp.float32)
    o_ref[...] = out[None]


def kernel(locations, levels, batch_indices, box_cls, box_regression,
           centerness, image_sizes):
    m, C = box_cls.shape
    N = image_sizes.shape[0]
    P = 128
    R = 2000

    bc = jnp.pad(box_cls.astype(jnp.float32), ((0, 0), (0, P - C)),
                 constant_values=-1e9)
    cn = centerness.astype(jnp.float32).reshape(m, 1)
    bi = batch_indices.astype(jnp.int32).reshape(m, 1)

    scored = pl.pallas_call(
        functools.partial(_score_body, C=C),
        grid=(N, m // R),
        in_specs=[
            pl.BlockSpec((R, P), lambda i, rb: (rb, 0)),
            pl.BlockSpec((R, 1), lambda i, rb: (rb, 0)),
            pl.BlockSpec((R, 1), lambda i, rb: (rb, 0)),
        ],
        out_specs=pl.BlockSpec((1, R, P), lambda i, rb: (i, rb, 0)),
        out_shape=jax.ShapeDtypeStruct((N, m, P), jnp.float32),
    )(bc, cn, bi)

    vals_k, idx_k = jax.lax.top_k(scored[:, :, :C].reshape(N, m * C),
                                  _PRE_NMS_TOP_N)
    rsel = idx_k // C
    csel = idx_k % C

    strides = jnp.asarray(_STRIDES, jnp.float32)
    reg = box_regression.astype(jnp.float32) * strides[levels][:, None]
    locs = locations.astype(jnp.float32)

    pad = _PADK - _PRE_NMS_TOP_N
    def padz(a, cv=0.0):
        return jnp.pad(a, ((0, 0), (0, pad)),
                       constant_values=cv).reshape(N, 1, _PADK)

    lx = padz(locs[rsel, 0])
    ly = padz(locs[rsel, 1])
    r0 = padz(reg[rsel, 0])
    r1 = padz(reg[rsel, 1])
    r2 = padz(reg[rsel, 2])
    r3 = padz(reg[rsel, 3])
    vp = padz(vals_k, cv=-1.0)
    clab = padz((csel + 1).astype(jnp.float32))
    hw = jnp.zeros((N, 1, 128), jnp.float32)
    hw = hw.at[:, 0, 0].set(image_sizes[:, 0].astype(jnp.float32))
    hw = hw.at[:, 0, 1].set(image_sizes[:, 1].astype(jnp.float32))

    vec = pl.BlockSpec((1, 1, _PADK), lambda i: (i, 0, 0))
    out = pl.pallas_call(
        _nms_body,
        grid=(N,),
        in_specs=[vec, vec, vec, vec, vec, vec, vec, vec,
                  pl.BlockSpec((1, 1, 128), lambda i: (i, 0, 0))],
        out_specs=pl.BlockSpec((1, 128, 8), lambda i: (i, 0, 0)),
        out_shape=jax.ShapeDtypeStruct((N, 128, 8), jnp.float32),
    )(lx, ly, r0, r1, r2, r3, vp, clab, hw)

    vals5 = out[:, :_POST_TOP_N, :5].reshape(N * _POST_TOP_N, 5)
    labels = jnp.rint(out[:, :_POST_TOP_N, 5]).astype(jnp.int32).reshape(-1)
    return vals5, labels
